# baseline (speedup 1.0000x reference)
import functools

import jax
import jax.numpy as jnp
from jax import lax
from jax.experimental import pallas as pl
from jax.experimental.pallas import tpu as pltpu
from jax.experimental.pallas import tpu_sc as plsc

_NUM_CORES = 2
_NUM_SUBCORES = 16
_NUM_WORKERS = _NUM_CORES * _NUM_SUBCORES
_LANES = 16
_JB = 8


@functools.lru_cache(maxsize=None)
def _make_sc_lookup(nj, ni, rows, d):
    w = ni // _NUM_WORKERS
    steps = nj // _JB
    units = (_JB * w) // _LANES
    cg = w // _LANES
    mesh = plsc.VectorSubcoreMesh(core_axis_name="c", subcore_axis_name="s")

    @functools.partial(
        pl.kernel,
        mesh=mesh,
        out_type=jax.ShapeDtypeStruct((nj, d, ni), jnp.float32),
        compiler_params=pltpu.CompilerParams(needs_layout_passes=False),
        scratch_types=[
            pltpu.VMEM((_JB, w), jnp.int32),
            pltpu.VMEM((_JB, w), jnp.int32),
            pltpu.VMEM((_JB, d, w), jnp.float32),
            pltpu.VMEM((_JB, d, w), jnp.float32),
            pltpu.VMEM((d, _LANES), jnp.float32),
            pltpu.SemaphoreType.DMA,
            pltpu.SemaphoreType.DMA,
            pltpu.SemaphoreType.DMA,
            pltpu.SemaphoreType.DMA,
        ],
    )
    def lookup(xt_hbm, table_hbm, out_hbm, idx0, idx1, rows0, rows1, tbl_v,
               si0, si1, so0, so1):
        wid = lax.axis_index("s") * _NUM_CORES + lax.axis_index("c")
        i0 = wid * w
        pltpu.sync_copy(table_hbm, tbl_v)
        idx_bufs, row_bufs = [idx0, idx1], [rows0, rows1]
        in_sems, out_sems = [si0, si1], [so0, so1]

        def start_in(t):
            return pltpu.async_copy(
                xt_hbm.at[pl.ds(t * _JB, _JB), pl.ds(i0, w)],
                idx_bufs[t % 2], in_sems[t % 2])

        def start_out(t):
            return pltpu.async_copy(
                row_bufs[t % 2],
                out_hbm.at[pl.ds(t * _JB, _JB), :, pl.ds(i0, w)],
                out_sems[t % 2])

        in_cp = [None] * steps
        out_cp = [None] * steps
        in_cp[0] = start_in(0)
        for t in range(steps):
            if t + 1 < steps:
                in_cp[t + 1] = start_in(t + 1)
            in_cp[t].wait()
            if t >= 2:
                out_cp[t - 2].wait()
            idx_v = idx_bufs[t % 2]
            rows_v = row_bufs[t % 2]

            @plsc.parallel_loop(0, units, 1, unroll=2)
            def _body(u):
                r = u // cg
                c = (u % cg) * _LANES
                v = idx_v[r, pl.ds(c, _LANES)]
                for k in range(d):
                    rows_v[r, k, pl.ds(c, _LANES)] = plsc.load_gather(
                        tbl_v.at[k], [v])

            out_cp[t] = start_out(t)
        for t in range(max(steps - 2, 0), steps):
            out_cp[t].wait()

    return lookup


def kernel(x, table):
    ni, nj = x.shape
    rows, d = table.shape
    tcols = jnp.zeros((d, _LANES), table.dtype).at[:, :rows].set(table.T)
    out_t = _make_sc_lookup(nj, ni, rows, d)(x.T.astype(jnp.int32), tcols)
    return out_t.transpose(2, 0, 1)


# speedup vs baseline: 1.0470x; 1.0470x over previous
import functools

import jax
import jax.numpy as jnp
from jax import lax
from jax.experimental import pallas as pl
from jax.experimental.pallas import tpu as pltpu
from jax.experimental.pallas import tpu_sc as plsc

_NUM_CORES = 2
_NUM_SUBCORES = 16
_NUM_WORKERS = _NUM_CORES * _NUM_SUBCORES
_LANES = 16
_JB = 8


@functools.lru_cache(maxsize=None)
def _make_sc_lookup(nj, ni, rows, d):
    w = ni // _NUM_WORKERS
    steps = nj // _JB
    units = (_JB * w) // _LANES
    cg = w // _LANES
    mesh = plsc.VectorSubcoreMesh(core_axis_name="c", subcore_axis_name="s")

    @functools.partial(
        pl.kernel,
        mesh=mesh,
        out_type=jax.ShapeDtypeStruct((nj, d, ni), jnp.float32),
        compiler_params=pltpu.CompilerParams(needs_layout_passes=False),
        scratch_types=[
            pltpu.VMEM((_JB, w), jnp.int32),
            pltpu.VMEM((_JB, w), jnp.int32),
            pltpu.VMEM((_JB, d, w), jnp.float32),
            pltpu.VMEM((_JB, d, w), jnp.float32),
            pltpu.VMEM((d, _LANES), jnp.float32),
            pltpu.SemaphoreType.DMA,
            pltpu.SemaphoreType.DMA,
            pltpu.SemaphoreType.DMA,
            pltpu.SemaphoreType.DMA,
        ],
    )
    def lookup(xt_hbm, table_hbm, out_hbm, idx0, idx1, rows0, rows1, tbl_v,
               si0, si1, so0, so1):
        wid = lax.axis_index("s") * _NUM_CORES + lax.axis_index("c")
        i0 = wid * w
        pltpu.sync_copy(table_hbm, tbl_v)
        idx_bufs, row_bufs = [idx0, idx1], [rows0, rows1]
        in_sems, out_sems = [si0, si1], [so0, so1]

        def start_in(t):
            return pltpu.async_copy(
                xt_hbm.at[pl.ds(t * _JB, _JB), pl.ds(i0, w)],
                idx_bufs[t % 2], in_sems[t % 2])

        def start_out(t):
            return pltpu.async_copy(
                row_bufs[t % 2],
                out_hbm.at[pl.ds(t * _JB, _JB), :, pl.ds(i0, w)],
                out_sems[t % 2])

        in_cp = [None] * steps
        out_cp = [None] * steps
        in_cp[0] = start_in(0)
        for t in range(steps):
            if t + 1 < steps:
                in_cp[t + 1] = start_in(t + 1)
            in_cp[t].wait()
            if t >= 2:
                out_cp[t - 2].wait()
            idx_v = idx_bufs[t % 2]
            rows_v = row_bufs[t % 2]

            @plsc.parallel_loop(0, units, 1, unroll=4)
            def _body(u):
                r = u // cg
                c = (u % cg) * _LANES
                v = idx_v[r, pl.ds(c, _LANES)]
                for k in range(d):
                    rows_v[r, k, pl.ds(c, _LANES)] = plsc.load_gather(
                        tbl_v.at[k], [v])

            out_cp[t] = start_out(t)
        for t in range(max(steps - 2, 0), steps):
            out_cp[t].wait()

    return lookup


def kernel(x, table):
    ni, nj = x.shape
    rows, d = table.shape
    tcols = jnp.zeros((d, _LANES), table.dtype).at[:, :rows].set(table.T)
    out_t = _make_sc_lookup(nj, ni, rows, d)(x.T.astype(jnp.int32), tcols)
    return out_t.transpose(2, 0, 1)


# speedup vs baseline: 1.0665x; 1.0186x over previous
import functools

import jax
import jax.numpy as jnp
from jax import lax
from jax.experimental import pallas as pl
from jax.experimental.pallas import tpu as pltpu
from jax.experimental.pallas import tpu_sc as plsc

_NUM_CORES = 2
_NUM_SUBCORES = 16
_NUM_WORKERS = _NUM_CORES * _NUM_SUBCORES
_LANES = 16
_JB = 8


@functools.lru_cache(maxsize=None)
def _make_sc_lookup(nj, ni, rows, d):
    w = ni // _NUM_WORKERS
    steps = nj // _JB
    units = (_JB * w) // _LANES
    cg = w // _LANES
    mesh = plsc.VectorSubcoreMesh(core_axis_name="c", subcore_axis_name="s")

    @functools.partial(
        pl.kernel,
        mesh=mesh,
        out_type=jax.ShapeDtypeStruct((nj, d, ni), jnp.float32),
        compiler_params=pltpu.CompilerParams(needs_layout_passes=False),
        scratch_types=[
            pltpu.VMEM((_JB, w), jnp.int32),
            pltpu.VMEM((_JB, w), jnp.int32),
            pltpu.VMEM((_JB, w), jnp.int32),
            pltpu.VMEM((_JB, d, w), jnp.float32),
            pltpu.VMEM((_JB, d, w), jnp.float32),
            pltpu.VMEM((_JB, d, w), jnp.float32),
            pltpu.VMEM((d, _LANES), jnp.float32),
            pltpu.SemaphoreType.DMA,
            pltpu.SemaphoreType.DMA,
            pltpu.SemaphoreType.DMA,
            pltpu.SemaphoreType.DMA,
            pltpu.SemaphoreType.DMA,
            pltpu.SemaphoreType.DMA,
        ],
    )
    def lookup(xt_hbm, table_hbm, out_hbm, idx0, idx1, idx2, rows0, rows1,
               rows2, tbl_v, si0, si1, si2, so0, so1, so2):
        wid = lax.axis_index("s") * _NUM_CORES + lax.axis_index("c")
        i0 = wid * w
        pltpu.sync_copy(table_hbm, tbl_v)
        idx_bufs, row_bufs = [idx0, idx1, idx2], [rows0, rows1, rows2]
        in_sems, out_sems = [si0, si1, si2], [so0, so1, so2]

        def start_in(t):
            return pltpu.async_copy(
                xt_hbm.at[pl.ds(t * _JB, _JB), pl.ds(i0, w)],
                idx_bufs[t % 3], in_sems[t % 3])

        def start_out(t):
            return pltpu.async_copy(
                row_bufs[t % 3],
                out_hbm.at[pl.ds(t * _JB, _JB), :, pl.ds(i0, w)],
                out_sems[t % 3])

        in_cp = [None] * steps
        out_cp = [None] * steps
        in_cp[0] = start_in(0)
        if steps > 1:
            in_cp[1] = start_in(1)
        for t in range(steps):
            if t + 2 < steps:
                in_cp[t + 2] = start_in(t + 2)
            in_cp[t].wait()
            if t >= 3:
                out_cp[t - 3].wait()
            idx_v = idx_bufs[t % 3]
            rows_v = row_bufs[t % 3]

            @plsc.parallel_loop(0, units, 1, unroll=4)
            def _body(u):
                r = u // cg
                c = (u % cg) * _LANES
                v = idx_v[r, pl.ds(c, _LANES)]
                for k in range(d):
                    rows_v[r, k, pl.ds(c, _LANES)] = plsc.load_gather(
                        tbl_v.at[k], [v])

            out_cp[t] = start_out(t)
        for t in range(max(steps - 3, 0), steps):
            out_cp[t].wait()

    return lookup


def kernel(x, table):
    ni, nj = x.shape
    rows, d = table.shape
    tcols = jnp.zeros((d, _LANES), table.dtype).at[:, :rows].set(table.T)
    out_t = _make_sc_lookup(nj, ni, rows, d)(x.T.astype(jnp.int32), tcols)
    return out_t.transpose(2, 0, 1)
